# baseline (speedup 1.0000x reference)
import functools

import jax
import jax.numpy as jnp
from jax import lax
from jax.experimental import pallas as pl
from jax.experimental.pallas import tpu as pltpu
from jax.experimental.pallas import tpu_sc as plsc

VOCAB = 1000000
DIM = 64
B = 4096
L = 200

NC = 2
NS = 16
NW = NC * NS
SEQ_PER_W = B // NW


GRP = 2
NGRP = SEQ_PER_W // GRP
GROWS = GRP * L
NBUF = 3


def _pool_body(idx_hbm, table_hbm, out_hbm, idx_v, buf0, buf1, buf2, out_v,
               sem0, sem1, sem2):
    c = lax.axis_index("c")
    s = lax.axis_index("s")
    wid = s * NC + c

    pltpu.sync_copy(idx_hbm.at[pl.ds(wid * SEQ_PER_W, SEQ_PER_W)], idx_v)

    bufs = ((buf0, sem0), (buf1, sem1), (buf2, sem2))

    def fire(g, j):
        buf, sem = bufs[j]
        for k in range(GRP):
            pltpu.async_copy(table_hbm.at[idx_v.at[GRP * g + k]],
                             buf.at[pl.ds(k * L, L)], sem)

    def drain(j):
        buf, sem = bufs[j]
        for k in range(GRP):
            pltpu.make_async_copy(table_hbm.at[idx_v.at[0]],
                                  buf.at[pl.ds(k * L, L)], sem).wait()

    def consume(g, j):
        buf, _ = bufs[j]
        drain(j)
        for k in range(GRP):
            seq = GRP * g + k
            base = k * L

            def acc_step(r, acc):
                r0 = base + 2 * r
                return tuple(
                    acc[i] + buf[r0 + i // 4, pl.ds(16 * (i % 4), 16)]
                    for i in range(8))

            zero = jnp.zeros((16,), jnp.float32)
            a = lax.fori_loop(0, L // 2, acc_step, (zero,) * 8)
            for d in range(4):
                out_v[seq, pl.ds(16 * d, 16)] = a[d] + a[d + 4]

    for j in range(NBUF):
        fire(j, j)

    def outer(go, carry):
        for j in range(NBUF):
            g = NBUF * go + j
            consume(g, j)

            @pl.when(g + NBUF < NGRP)
            def _():
                fire(g + NBUF, j)
        return carry

    lax.fori_loop(0, NGRP // NBUF, outer, 0)
    consume(NGRP - 1, (NGRP - 1) % NBUF)
    pltpu.sync_copy(out_v, out_hbm.at[pl.ds(wid * SEQ_PER_W, SEQ_PER_W)])


@functools.partial(jax.jit, static_argnames=())
def _sc_pool(idx_arr, emb_table):
    mesh = plsc.VectorSubcoreMesh(core_axis_name="c", subcore_axis_name="s")
    return pl.kernel(
        _pool_body,
        mesh=mesh,
        compiler_params=pltpu.CompilerParams(use_tc_tiling_on_sc=False),
        out_type=jax.ShapeDtypeStruct((B, DIM), jnp.float32),
        scratch_types=[
            pltpu.VMEM((SEQ_PER_W, L), jnp.int32),
            pltpu.VMEM((GROWS, DIM), jnp.float32),
            pltpu.VMEM((GROWS, DIM), jnp.float32),
            pltpu.VMEM((GROWS, DIM), jnp.float32),
            pltpu.VMEM((SEQ_PER_W, DIM), jnp.float32),
            pltpu.SemaphoreType.DMA,
            pltpu.SemaphoreType.DMA,
            pltpu.SemaphoreType.DMA,
        ],
    )(idx_arr, emb_table)


def _mm_body(x_ref, wt_ref, b_ref, o_ref):
    o_ref[...] = jnp.dot(x_ref[...], wt_ref[...],
                         preferred_element_type=jnp.float32) + b_ref[...]


def _tc_linear(pooled, wt, b2d):
    return pl.pallas_call(
        _mm_body,
        out_shape=jax.ShapeDtypeStruct((B, DIM), jnp.float32),
    )(pooled, wt, b2d)


def kernel(tokens, emb_table, W, b):
    pooled = _sc_pool(tokens, emb_table)
    wt = (W.T * (1.0 / L)).astype(jnp.float32)
    return _tc_linear(pooled, wt, b.reshape(1, DIM))


# speedup vs baseline: 1.0057x; 1.0057x over previous
import functools

import jax
import jax.numpy as jnp
from jax import lax
from jax.experimental import pallas as pl
from jax.experimental.pallas import tpu as pltpu
from jax.experimental.pallas import tpu_sc as plsc

VOCAB = 1000000
DIM = 64
B = 4096
L = 200

NC = 2
NS = 16
NW = NC * NS
SEQ_PER_W = B // NW


GRP = 2
NGRP = SEQ_PER_W // GRP
GROWS = GRP * L
NBUF = 3


def _pool_body(idx_hbm, table_hbm, out_hbm, idx_v, buf0, buf1, buf2, out_v,
               sem0, sem1, sem2):
    c = lax.axis_index("c")
    s = lax.axis_index("s")
    wid = s * NC + c

    pltpu.sync_copy(idx_hbm.at[pl.ds(wid * SEQ_PER_W * L, SEQ_PER_W * L)],
                    idx_v)

    bufs = ((buf0, sem0), (buf1, sem1), (buf2, sem2))

    def fire(g, j):
        buf, sem = bufs[j]
        pltpu.async_copy(table_hbm.at[idx_v.at[pl.ds(g * GROWS, GROWS)]],
                         buf, sem)

    def drain(j):
        buf, sem = bufs[j]
        pltpu.make_async_copy(table_hbm.at[idx_v.at[pl.ds(0, GROWS)]],
                              buf, sem).wait()

    def consume(g, j):
        buf, _ = bufs[j]
        drain(j)
        for k in range(GRP):
            seq = GRP * g + k
            base = k * L

            def acc_step(r, acc):
                r0 = base + 2 * r
                return tuple(
                    acc[i] + buf[r0 + i // 4, pl.ds(16 * (i % 4), 16)]
                    for i in range(8))

            zero = jnp.zeros((16,), jnp.float32)
            a = lax.fori_loop(0, L // 2, acc_step, (zero,) * 8)
            for d in range(4):
                out_v[seq, pl.ds(16 * d, 16)] = a[d] + a[d + 4]

    for j in range(NBUF):
        fire(j, j)

    def outer(go, carry):
        for j in range(NBUF):
            g = NBUF * go + j
            consume(g, j)

            @pl.when(g + NBUF < NGRP)
            def _():
                fire(g + NBUF, j)
        return carry

    lax.fori_loop(0, NGRP // NBUF, outer, 0)
    consume(NGRP - 1, (NGRP - 1) % NBUF)
    pltpu.sync_copy(out_v, out_hbm.at[pl.ds(wid * SEQ_PER_W, SEQ_PER_W)])


@functools.partial(jax.jit, static_argnames=())
def _sc_pool(idx_arr, emb_table):
    mesh = plsc.VectorSubcoreMesh(core_axis_name="c", subcore_axis_name="s")
    return pl.kernel(
        _pool_body,
        mesh=mesh,
        compiler_params=pltpu.CompilerParams(use_tc_tiling_on_sc=False),
        out_type=jax.ShapeDtypeStruct((B, DIM), jnp.float32),
        scratch_types=[
            pltpu.VMEM((SEQ_PER_W * L,), jnp.int32),
            pltpu.VMEM((GROWS, DIM), jnp.float32),
            pltpu.VMEM((GROWS, DIM), jnp.float32),
            pltpu.VMEM((GROWS, DIM), jnp.float32),
            pltpu.VMEM((SEQ_PER_W, DIM), jnp.float32),
            pltpu.SemaphoreType.DMA,
            pltpu.SemaphoreType.DMA,
            pltpu.SemaphoreType.DMA,
        ],
    )(idx_arr, emb_table)


def _mm_body(x_ref, wt_ref, b_ref, o_ref):
    o_ref[...] = jnp.dot(x_ref[...], wt_ref[...],
                         preferred_element_type=jnp.float32) + b_ref[...]


def _tc_linear(pooled, wt, b2d):
    return pl.pallas_call(
        _mm_body,
        out_shape=jax.ShapeDtypeStruct((B, DIM), jnp.float32),
    )(pooled, wt, b2d)


def kernel(tokens, emb_table, W, b):
    pooled = _sc_pool(tokens.reshape(B * L), emb_table)
    wt = (W.T * (1.0 / L)).astype(jnp.float32)
    return _tc_linear(pooled, wt, b.reshape(1, DIM))
